# Initial kernel scaffold; baseline (speedup 1.0000x reference)
import jax
import jax.numpy as jnp
from jax.experimental import pallas as pl


def kernel(x, W_enc, W_dec, b_enc, b_dec, gate_raw):
    raise NotImplementedError("write your pallas kernel here")


# speedup vs baseline: 8.3979x; 8.3979x over previous
import jax
import jax.numpy as jnp
from jax import lax
from jax.experimental import pallas as pl
from jax.experimental.pallas import tpu as pltpu

_K = 128


def _i32_min():
    return jnp.int32(-(2 ** 31))


def _encode_body(x_ref, w_ref, b_ref, out_ref):
    out_ref[...] = (
        jnp.dot(x_ref[...], w_ref[...], preferred_element_type=jnp.float32)
        + b_ref[...]
    )


def _monotone_key(v):
    b = lax.bitcast_convert_type(v, jnp.int32)
    return jnp.where(b >= 0, b, _i32_min() - b)


def _topk_mask(pre, k):
    s = _monotone_key(pre)

    def body(i, m_u):
        bit = 31 - i
        cand_u = m_u | (jnp.int32(1) << bit)
        cand_s = cand_u ^ _i32_min()
        cnt = jnp.sum((s >= cand_s).astype(jnp.int32), axis=1, keepdims=True)
        return jnp.where(cnt >= k, cand_u, m_u)

    m_u = lax.fori_loop(0, 32, body, jnp.zeros((pre.shape[0], 1), jnp.int32))
    tau_s = m_u ^ _i32_min()
    return jnp.where(s >= tau_s, jnp.maximum(pre, 0.0), 0.0)


def _recur_body(pre_ref, gate_ref, z_ref):
    B, T, S = pre_ref.shape
    gate = gate_ref[...]
    zprev = None
    for t in range(T):
        pre = pre_ref[:, t, :]
        if t > 0:
            pre = gate * zprev + pre
        z = _topk_mask(pre, _K)
        z_ref[:, t, :] = z
        zprev = z


def _decode_body(z_ref, w_ref, b_ref, x_ref, xhat_ref, loss_ref, *, nk, inv_bt):
    k = pl.program_id(0)
    part = jnp.dot(z_ref[...], w_ref[...], preferred_element_type=jnp.float32)

    @pl.when(k == 0)
    def _():
        xhat_ref[...] = part

    @pl.when(k > 0)
    def _():
        xhat_ref[...] = xhat_ref[...] + part

    @pl.when(k == nk - 1)
    def _():
        xh = xhat_ref[...] + b_ref[...]
        xhat_ref[...] = xh
        d = xh - x_ref[...]
        loss_ref[0, 0] = jnp.sum(d * d) * inv_bt


def kernel(x, W_enc, W_dec, b_enc, b_dec, gate_raw):
    B, T, D_IN = x.shape
    D_SAE = W_enc.shape[1]
    BT = B * T

    x2 = x.reshape(BT, D_IN)
    gate = jax.nn.sigmoid(gate_raw).reshape(1, D_SAE)
    b_enc2 = b_enc.reshape(1, D_SAE)
    b_dec2 = b_dec.reshape(1, D_IN)

    SN = 2048
    pre2 = pl.pallas_call(
        _encode_body,
        grid=(D_SAE // SN,),
        in_specs=[
            pl.BlockSpec((BT, D_IN), lambda j: (0, 0)),
            pl.BlockSpec((D_IN, SN), lambda j: (0, j)),
            pl.BlockSpec((1, SN), lambda j: (0, j)),
        ],
        out_specs=pl.BlockSpec((BT, SN), lambda j: (0, j)),
        out_shape=jax.ShapeDtypeStruct((BT, D_SAE), jnp.float32),
    )(x2, W_enc, b_enc2)

    z_seq = pl.pallas_call(
        _recur_body,
        in_specs=[
            pl.BlockSpec((B, T, D_SAE), lambda: (0, 0, 0)),
            pl.BlockSpec((1, D_SAE), lambda: (0, 0)),
        ],
        out_specs=pl.BlockSpec((B, T, D_SAE), lambda: (0, 0, 0)),
        out_shape=jax.ShapeDtypeStruct((B, T, D_SAE), jnp.float32),
    )(pre2.reshape(B, T, D_SAE), gate)

    SK = 1024
    NK = D_SAE // SK
    import functools

    xhat2, loss = pl.pallas_call(
        functools.partial(_decode_body, nk=NK, inv_bt=1.0 / BT),
        grid=(NK,),
        in_specs=[
            pl.BlockSpec((BT, SK), lambda k: (0, k)),
            pl.BlockSpec((SK, D_IN), lambda k: (k, 0)),
            pl.BlockSpec((1, D_IN), lambda k: (0, 0)),
            pl.BlockSpec((BT, D_IN), lambda k: (0, 0)),
        ],
        out_specs=[
            pl.BlockSpec((BT, D_IN), lambda k: (0, 0)),
            pl.BlockSpec(memory_space=pltpu.SMEM),
        ],
        out_shape=[
            jax.ShapeDtypeStruct((BT, D_IN), jnp.float32),
            jax.ShapeDtypeStruct((1, 1), jnp.float32),
        ],
    )(z_seq.reshape(BT, D_SAE), W_dec, b_dec2, x2)

    return (loss[0, 0], xhat2.reshape(B, T, D_IN), z_seq[:, -1, :])
